# Initial kernel scaffold; baseline (speedup 1.0000x reference)
import jax
import jax.numpy as jnp
from jax.experimental import pallas as pl


def kernel(X, Y):
    raise NotImplementedError("write your pallas kernel here")


# speedup vs baseline: 4.0876x; 4.0876x over previous
import functools

import jax
import jax.numpy as jnp
from jax.experimental import pallas as pl
from jax.experimental.pallas import tpu as pltpu

_K0 = 0.11831801
_K1 = 0.14776132

_CROP = 5


def _loss_body(x_ref, y_ref, o_ref, *, w_blk, H, W):
    j = pl.program_id(1)
    n = H - 2 * _CROP
    x = x_ref[0, 0, 4:H - 4, :]
    y = y_ref[0, 0, 4:H - 4, :]

    def conv(a):
        return _K0 * (a[0:n] + a[2:n + 2]) + _K1 * a[1:n + 1]

    mu1 = conv(x)
    mu2 = conv(y)
    s11 = conv(x * x) - mu1 * mu1
    s22 = conv(y * y) - mu2 * mu2
    s12 = conv(x * y) - mu1 * mu2
    loss = s11 * s22 - 2.0 * s12

    col = j * w_blk + jax.lax.broadcasted_iota(jnp.int32, loss.shape, 1)
    loss = jnp.where((col >= _CROP) & (col < W - _CROP), loss, 0.0)

    s = jnp.sum(loss) * (1.0 / 128.0)
    o_ref[0, 0, :] = jnp.full((128,), s, dtype=jnp.float32)


def kernel(X, Y):
    B, C, H, W = X.shape
    w_blk = 512 if W % 512 == 0 else W
    nj = W // w_blk

    out = pl.pallas_call(
        functools.partial(_loss_body, w_blk=w_blk, H=H, W=W),
        out_shape=jax.ShapeDtypeStruct((B, 1, nj * 128), jnp.float32),
        grid=(B, nj),
        in_specs=[
            pl.BlockSpec((1, 1, H, w_blk), lambda b, j: (b, 0, 0, j)),
            pl.BlockSpec((1, 1, H, w_blk), lambda b, j: (b, 0, 0, j)),
        ],
        out_specs=pl.BlockSpec((1, 1, 128), lambda b, j: (b, 0, j)),
        compiler_params=pltpu.CompilerParams(
            dimension_semantics=("parallel", "parallel"),
        ),
        name="ssim_loss",
    )(X, Y)

    n = jnp.float32(H - 2 * _CROP) * jnp.float32(W - 2 * _CROP)
    return jnp.sum(out) / n
